# baseline (speedup 1.0000x reference)
import functools

import jax
import jax.numpy as jnp
from jax import lax
from jax.experimental import pallas as pl
from jax.experimental.pallas import tpu as pltpu
from jax.experimental.pallas import tpu_sc as plsc

_V, _D = 1_000_000, 32
_B = 1_048_576

_NC, _NS = 2, 16
_NW = _NC * _NS
_BPW = _B // _NW
_CHUNK = 1024
_NCHUNK = _BPW // _CHUNK
_NPAIR = _NCHUNK // 2
_L = 16


def _body(table_hbm, idx_hbm, outT_hbm, i0, i1, g0, g1, tb,
          gs0, gs1, ws):
    wid = lax.axis_index("s") * _NC + lax.axis_index("c")
    base = wid * _BPW

    ibuf = (i0, i1)
    gbuf = (g0, g1)
    gsem = (gs0, gs1)

    base0 = lax.iota(jnp.int32, _L) * _CHUNK
    base1 = base0 + _L * _CHUNK

    def load_idx(t, b):
        pltpu.sync_copy(idx_hbm.at[pl.ds(base + t * _CHUNK, _CHUNK)], ibuf[b])

    def start_gather(b):
        pltpu.async_copy(table_hbm.at[ibuf[b]], gbuf[b], gsem[b])

    def wait_gather(b):
        pltpu.make_async_copy(table_hbm.at[ibuf[b]], gbuf[b], gsem[b]).wait()

    def start_writes(t, b):
        col = base + t * _CHUNK
        for c in range(_D):
            pltpu.async_copy(tb.at[pl.ds(c * _CHUNK, _CHUNK)],
                             outT_hbm.at[c, pl.ds(col, _CHUNK)], ws)

    def drain_writes():
        for c in range(_D):
            pltpu.make_async_copy(tb.at[pl.ds(c * _CHUNK, _CHUNK)],
                                  outT_hbm.at[c, pl.ds(0, _CHUNK)], ws).wait()

    def transpose(b):
        def row(i, carry):
            x0 = gbuf[b][i, pl.ds(0, _L)]
            x1 = gbuf[b][i, pl.ds(_L, _L)]
            plsc.store_scatter(tb, [base0 + i], x0)
            plsc.store_scatter(tb, [base1 + i], x1)
            return carry
        lax.fori_loop(0, _CHUNK, row, 0, unroll=8)

    load_idx(0, 0)
    load_idx(1, 1)
    start_gather(0)
    start_gather(1)

    def pair(g, carry):
        for b in (0, 1):
            t = 2 * g + b
            wait_gather(b)

            @pl.when(t > 0)
            def _():
                drain_writes()

            transpose(b)
            start_writes(t, b)

            @pl.when(g < _NPAIR - 1)
            def _():
                load_idx(t + 2, b)
                start_gather(b)
        return carry

    lax.fori_loop(0, _NPAIR, pair, 0)
    drain_writes()


@functools.lru_cache(maxsize=1)
def _build():
    mesh = plsc.VectorSubcoreMesh(core_axis_name="c", subcore_axis_name="s")
    return pl.kernel(
        _body,
        mesh=mesh,
        out_type=jax.ShapeDtypeStruct((_D, _B), jnp.float32),
        scratch_types=[
            pltpu.VMEM((_CHUNK,), jnp.int32),
            pltpu.VMEM((_CHUNK,), jnp.int32),
            pltpu.VMEM((_CHUNK, _D), jnp.float32),
            pltpu.VMEM((_CHUNK, _D), jnp.float32),
            pltpu.VMEM((_D * _CHUNK,), jnp.float32),
            pltpu.SemaphoreType.DMA,
            pltpu.SemaphoreType.DMA,
            pltpu.SemaphoreType.DMA,
        ],
        compiler_params=pltpu.CompilerParams(
            use_tc_tiling_on_sc=False, needs_layout_passes=False),
    )


def kernel(features, rev):
    outT = _build()(features, rev.astype(jnp.int32))
    return outT.T


# speedup vs baseline: 5.2428x; 5.2428x over previous
import functools

import jax
import jax.numpy as jnp
from jax import lax
from jax.experimental import pallas as pl
from jax.experimental.pallas import tpu as pltpu
from jax.experimental.pallas import tpu_sc as plsc

_V, _D = 1_000_000, 32
_B = 1_048_576
_DP = 128

_NC, _NS = 2, 16
_NW = _NC * _NS
_BPW = _B // _NW
_CHUNK = 1024
_NCHUNK = _BPW // _CHUNK
_NBUF = 2


def _body(table_hbm, idx_hbm, out_hbm, idx_all, rows0, rows1, gs0, gs1, ws0, ws1):
    wid = lax.axis_index("s") * _NC + lax.axis_index("c")
    base = wid * _BPW

    pltpu.sync_copy(idx_hbm.at[pl.ds(base, _BPW)], idx_all)

    rows = (rows0, rows1)
    gsem = (gs0, gs1)
    wsem = (ws0, ws1)
    gd = [None] * _NCHUNK
    wd = [None] * _NCHUNK
    for c in range(_NCHUNK):
        b = c % _NBUF
        if c >= _NBUF:
            wd[c - _NBUF].wait()
        gd[c] = pltpu.async_copy(
            table_hbm.at[idx_all.at[pl.ds(c * _CHUNK, _CHUNK)]], rows[b], gsem[b])
        if c >= 1:
            bp = (c - 1) % _NBUF
            gd[c - 1].wait()
            wd[c - 1] = pltpu.async_copy(
                rows[bp],
                out_hbm.at[pl.ds(base + (c - 1) * _CHUNK, _CHUNK), pl.ds(0, _D)],
                wsem[bp])
    last = _NCHUNK - 1
    gd[last].wait()
    wd[last] = pltpu.async_copy(
        rows[last % _NBUF],
        out_hbm.at[pl.ds(base + last * _CHUNK, _CHUNK), pl.ds(0, _D)],
        wsem[last % _NBUF])
    wd[last - 1].wait()
    wd[last].wait()


@functools.lru_cache(maxsize=1)
def _build():
    mesh = plsc.VectorSubcoreMesh(core_axis_name="c", subcore_axis_name="s")
    return pl.kernel(
        _body,
        mesh=mesh,
        out_type=jax.ShapeDtypeStruct((_B, _DP), jnp.float32),
        scratch_types=[
            pltpu.VMEM((_BPW,), jnp.int32),
            pltpu.VMEM((_CHUNK, _D), jnp.float32),
            pltpu.VMEM((_CHUNK, _D), jnp.float32),
            pltpu.SemaphoreType.DMA,
            pltpu.SemaphoreType.DMA,
            pltpu.SemaphoreType.DMA,
            pltpu.SemaphoreType.DMA,
        ],
        compiler_params=pltpu.CompilerParams(
            use_tc_tiling_on_sc=False, needs_layout_passes=False),
    )


def kernel(features, rev):
    out128 = _build()(features, rev.astype(jnp.int32))
    return out128[:, :_D]
